# Initial kernel scaffold; baseline (speedup 1.0000x reference)
import jax
import jax.numpy as jnp
from jax.experimental import pallas as pl


def kernel(embeddings, slot_embeddings, _input):
    raise NotImplementedError("write your pallas kernel here")


# speedup vs baseline: 2.1174x; 2.1174x over previous
import functools

import jax
import jax.numpy as jnp
from jax import lax
from jax.experimental import pallas as pl
from jax.experimental.pallas import tpu as pltpu
from jax.experimental.pallas import tpu_sc as plsc

NUM_CORES = 2
NUM_SUBCORES = 16
NUM_WORKERS = NUM_CORES * NUM_SUBCORES
LANES = 16
GROUP = 128
SLOT_START = 49


def _build(num_tokens, dim, pad_rows):
  tok_per_tile = num_tokens // NUM_WORKERS
  num_groups = tok_per_tile // GROUP
  assert tok_per_tile * NUM_WORKERS == num_tokens
  assert num_groups * GROUP == tok_per_tile
  assert dim % LANES == 0

  mesh = plsc.VectorSubcoreMesh(
      core_axis_name="c", subcore_axis_name="s",
      num_cores=NUM_CORES, num_subcores=NUM_SUBCORES)

  @functools.partial(
      pl.kernel,
      out_type=jax.ShapeDtypeStruct((num_tokens + pad_rows, dim), jnp.float32),
      mesh=mesh,
      scratch_types=[
          pltpu.VMEM((GROUP,), jnp.int32),
          pltpu.VMEM((GROUP, dim), jnp.float32),
          pltpu.VMEM((GROUP,), jnp.int32),
          pltpu.VMEM((GROUP,), jnp.int32),
          pltpu.VMEM((GROUP, dim), jnp.float32),
          pltpu.SemaphoreType.DMA,
      ],
      compiler_params=pltpu.CompilerParams(needs_layout_passes=False),
  )
  def gather_kernel(emb, slot, idx_hbm, out, gidx, rows, sidx, spos, frows,
                    sem):
    wid = lax.axis_index("s") * NUM_CORES + lax.axis_index("c")
    base = wid * tok_per_tile
    dump_row = num_tokens + wid
    lane = lax.broadcasted_iota(jnp.int32, (LANES,), 0)

    def group_body(g, _):
      i0 = base + g * GROUP
      pltpu.sync_copy(idx_hbm.at[pl.ds(i0, GROUP)], gidx)
      gcp = pltpu.async_copy(emb.at[gidx], rows, sem)
      m_any = gidx[pl.ds(0, LANES)] <= SLOT_START
      for j in range(1, GROUP // LANES):
        m_any = m_any | (gidx[pl.ds(j * LANES, LANES)] <= SLOT_START)
      has_slot = jnp.any(m_any)

      gcp.wait()
      pltpu.sync_copy(rows, out.at[pl.ds(i0, GROUP)])

      @pl.when(has_slot)
      def _fixup():
        for j in range(GROUP // LANES):
          v = gidx[pl.ds(j * LANES, LANES)]
          m = v <= SLOT_START
          sidx[pl.ds(j * LANES, LANES)] = jnp.where(m, SLOT_START - v, 0)
          spos[pl.ds(j * LANES, LANES)] = jnp.where(
              m, i0 + j * LANES + lane, dump_row)
        pltpu.sync_copy(slot.at[sidx], frows)
        pltpu.sync_copy(frows, out.at[spos])

      return ()

    lax.fori_loop(0, num_groups, group_body, (), unroll=False)

  return gather_kernel


def kernel(embeddings, slot_embeddings, _input):
  batch, seq = _input.shape
  dim = embeddings.shape[1]
  num_tokens = batch * seq
  pad_rows = NUM_WORKERS
  idx = _input.reshape(num_tokens)
  out = _build(num_tokens, dim, pad_rows)(embeddings, slot_embeddings, idx)
  return out[:num_tokens].reshape(batch, seq, dim)


# speedup vs baseline: 2.6293x; 1.2418x over previous
import functools

import jax
import jax.numpy as jnp
from jax import lax
from jax.experimental import pallas as pl
from jax.experimental.pallas import tpu as pltpu
from jax.experimental.pallas import tpu_sc as plsc

NUM_CORES = 2
NUM_SUBCORES = 16
NUM_WORKERS = NUM_CORES * NUM_SUBCORES
LANES = 16
GROUP = 128
NBUF = 4
SLOT_START = 49


def _build(num_tokens, dim, pad_rows):
  tok_per_tile = num_tokens // NUM_WORKERS
  num_groups = tok_per_tile // GROUP
  assert tok_per_tile * NUM_WORKERS == num_tokens
  assert num_groups * GROUP == tok_per_tile
  assert num_groups % NBUF == 0 and num_groups // NBUF >= 2
  assert dim % LANES == 0

  mesh = plsc.VectorSubcoreMesh(
      core_axis_name="c", subcore_axis_name="s",
      num_cores=NUM_CORES, num_subcores=NUM_SUBCORES)

  @functools.partial(
      pl.kernel,
      out_type=jax.ShapeDtypeStruct((num_tokens + pad_rows, dim), jnp.float32),
      mesh=mesh,
      scratch_types=[
          pltpu.VMEM((tok_per_tile,), jnp.int32),
          pltpu.VMEM((NBUF, GROUP, dim), jnp.float32),
          pltpu.VMEM((GROUP,), jnp.int32),
          pltpu.VMEM((GROUP,), jnp.int32),
          pltpu.VMEM((GROUP, dim), jnp.float32),
          [pltpu.SemaphoreType.DMA] * NBUF,
          [pltpu.SemaphoreType.DMA] * NBUF,
      ],
      compiler_params=pltpu.CompilerParams(needs_layout_passes=False),
  )
  def gather_kernel(emb, slot, idx_hbm, out, idx_v, rows, sidx, spos, frows,
                    sem_g, sem_w):
    wid = lax.axis_index("s") * NUM_CORES + lax.axis_index("c")
    base = wid * tok_per_tile
    dump_row = num_tokens + wid
    lane = lax.broadcasted_iota(jnp.int32, (LANES,), 0)

    pltpu.sync_copy(idx_hbm.at[pl.ds(base, tok_per_tile)], idx_v)

    def start_gather(g, b):
      pltpu.async_copy(
          emb.at[idx_v.at[pl.ds(g * GROUP, GROUP)]], rows.at[b], sem_g[b])

    def wait_gather(b):
      pltpu.make_async_copy(
          emb.at[pl.ds(0, GROUP)], rows.at[b], sem_g[b]).wait()

    for b in range(NBUF):
      start_gather(b, b)

    def ring_body(k, _):
      for b in range(NBUF):
        g = k * NBUF + b
        wait_gather(b)
        wcp = pltpu.async_copy(
            rows.at[b], out.at[pl.ds(base + g * GROUP, GROUP)], sem_w[b])
        wcp.wait()
        start_gather(g + NBUF, b)
      return ()

    lax.fori_loop(0, num_groups // NBUF - 1, ring_body, (), unroll=False)

    for b in range(NBUF):
      g = num_groups - NBUF + b
      wait_gather(b)
      pltpu.sync_copy(rows.at[b], out.at[pl.ds(base + g * GROUP, GROUP)])

    def fixup_body(g, _):
      loff = g * GROUP
      m_any = idx_v[pl.ds(loff, LANES)] <= SLOT_START
      for j in range(1, GROUP // LANES):
        m_any = m_any | (idx_v[pl.ds(loff + j * LANES, LANES)] <= SLOT_START)

      @pl.when(jnp.any(m_any))
      def _fixup():
        for j in range(GROUP // LANES):
          v = idx_v[pl.ds(loff + j * LANES, LANES)]
          m = v <= SLOT_START
          sidx[pl.ds(j * LANES, LANES)] = jnp.where(m, SLOT_START - v, 0)
          spos[pl.ds(j * LANES, LANES)] = jnp.where(
              m, base + loff + j * LANES + lane, dump_row)
        pltpu.sync_copy(slot.at[sidx], frows)
        pltpu.sync_copy(frows, out.at[spos])

      return ()

    lax.fori_loop(0, num_groups, fixup_body, (), unroll=False)

  return gather_kernel


def kernel(embeddings, slot_embeddings, _input):
  batch, seq = _input.shape
  dim = embeddings.shape[1]
  num_tokens = batch * seq
  pad_rows = NUM_WORKERS
  idx = _input.reshape(num_tokens)
  out = _build(num_tokens, dim, pad_rows)(embeddings, slot_embeddings, idx)
  return out[:num_tokens].reshape(batch, seq, dim)


# speedup vs baseline: 2.6298x; 1.0002x over previous
import functools

import jax
import jax.numpy as jnp
from jax import lax
from jax.experimental import pallas as pl
from jax.experimental.pallas import tpu as pltpu
from jax.experimental.pallas import tpu_sc as plsc

NUM_CORES = 2
NUM_SUBCORES = 16
NUM_WORKERS = NUM_CORES * NUM_SUBCORES
LANES = 16
GROUP = 128
NBUF = 5
SLOT_START = 49


def _build(num_tokens, dim, pad_rows):
  tok_per_tile = num_tokens // NUM_WORKERS
  num_groups = tok_per_tile // GROUP
  assert tok_per_tile * NUM_WORKERS == num_tokens
  assert num_groups * GROUP == tok_per_tile
  assert num_groups % NBUF == 0 and num_groups // NBUF >= 2
  assert dim % LANES == 0

  mesh = plsc.VectorSubcoreMesh(
      core_axis_name="c", subcore_axis_name="s",
      num_cores=NUM_CORES, num_subcores=NUM_SUBCORES)

  @functools.partial(
      pl.kernel,
      out_type=jax.ShapeDtypeStruct((num_tokens + pad_rows, dim), jnp.float32),
      mesh=mesh,
      scratch_types=[
          pltpu.VMEM((tok_per_tile,), jnp.int32),
          pltpu.VMEM((NBUF, GROUP, dim), jnp.float32),
          pltpu.VMEM((GROUP,), jnp.int32),
          pltpu.VMEM((GROUP,), jnp.int32),
          pltpu.VMEM((GROUP, dim), jnp.float32),
          [pltpu.SemaphoreType.DMA] * NBUF,
          [pltpu.SemaphoreType.DMA] * NBUF,
      ],
      compiler_params=pltpu.CompilerParams(needs_layout_passes=False),
  )
  def gather_kernel(emb, slot, idx_hbm, out, idx_v, rows, sidx, spos, frows,
                    sem_g, sem_w):
    wid = lax.axis_index("s") * NUM_CORES + lax.axis_index("c")
    base = wid * tok_per_tile
    dump_row = num_tokens + wid
    lane = lax.broadcasted_iota(jnp.int32, (LANES,), 0)

    pltpu.sync_copy(idx_hbm.at[pl.ds(base, tok_per_tile)], idx_v)

    def start_gather(g, b):
      pltpu.async_copy(
          emb.at[idx_v.at[pl.ds(g * GROUP, GROUP)]], rows.at[b], sem_g[b])

    def wait_gather(b):
      pltpu.make_async_copy(
          emb.at[pl.ds(0, GROUP)], rows.at[b], sem_g[b]).wait()

    def start_write(g, b):
      pltpu.async_copy(
          rows.at[b], out.at[pl.ds(base + g * GROUP, GROUP)], sem_w[b])

    def wait_write(b):
      pltpu.make_async_copy(
          rows.at[b], out.at[pl.ds(base, GROUP)], sem_w[b]).wait()

    for g in range(NBUF - 1):
      start_gather(g, g)

    wait_gather(0)
    start_write(0, 0)
    start_gather(NBUF - 1, NBUF - 1)

    def ring_block(k, first, last):
      def step(g, b):
        wait_gather(b)
        start_write(g, b)
        bp = (b - 1) % NBUF
        wait_write(bp)
        if not last or b == 0:
          start_gather(g + NBUF - 1, bp)

      for b in range(NBUF):
        g = k * NBUF + b
        if first and b == 0:
          continue
        if last and b > 0:
          wait_gather(b)
          start_write(g, b)
          wait_write(b - 1)
        else:
          step(g, b)

    ring_block(0, True, False)
    lax.fori_loop(
        1, num_groups // NBUF - 1,
        lambda k, _: (ring_block(k, False, False), ())[1], (), unroll=False)
    ring_block(num_groups // NBUF - 1, False, True)
    wait_write(NBUF - 1)

    def fixup_body(g, _):
      loff = g * GROUP
      m_any = idx_v[pl.ds(loff, LANES)] <= SLOT_START
      for j in range(1, GROUP // LANES):
        m_any = m_any | (idx_v[pl.ds(loff + j * LANES, LANES)] <= SLOT_START)

      @pl.when(jnp.any(m_any))
      def _fixup():
        for j in range(GROUP // LANES):
          v = idx_v[pl.ds(loff + j * LANES, LANES)]
          m = v <= SLOT_START
          sidx[pl.ds(j * LANES, LANES)] = jnp.where(m, SLOT_START - v, 0)
          spos[pl.ds(j * LANES, LANES)] = jnp.where(
              m, base + loff + j * LANES + lane, dump_row)
        pltpu.sync_copy(slot.at[sidx], frows)
        pltpu.sync_copy(frows, out.at[spos])

      return ()

    lax.fori_loop(0, num_groups, fixup_body, (), unroll=False)

  return gather_kernel


def kernel(embeddings, slot_embeddings, _input):
  batch, seq = _input.shape
  dim = embeddings.shape[1]
  num_tokens = batch * seq
  pad_rows = NUM_WORKERS
  idx = _input.reshape(num_tokens)
  out = _build(num_tokens, dim, pad_rows)(embeddings, slot_embeddings, idx)
  return out[:num_tokens].reshape(batch, seq, dim)
